# baseline (speedup 1.0000x reference)
import functools

import jax
import jax.numpy as jnp
from jax import lax
from jax.experimental import pallas as pl
from jax.experimental.pallas import tpu as pltpu
from jax.experimental.pallas import tpu_sc as plsc

_SC_WORKERS = 32
_GATHER_CHUNK = 80


def _sc_gather(table, col):
    E = col.shape[0]
    D = table.shape[1]
    dt = table.dtype
    e_per_w = E // _SC_WORKERS
    NB = 5
    ch = next(c for c in (_GATHER_CHUNK, 40, 8) if e_per_w % (c * NB) == 0)
    n_ch = e_per_w // ch
    mesh = plsc.VectorSubcoreMesh(core_axis_name="c", subcore_axis_name="s")

    n_outer = n_ch // NB

    @functools.partial(
        pl.kernel,
        out_type=jax.ShapeDtypeStruct((E, D), dt),
        mesh=mesh,
        compiler_params=pltpu.CompilerParams(use_tc_tiling_on_sc=False),
        scratch_types=[
            pltpu.VMEM((NB, ch), jnp.int32),
            pltpu.VMEM((NB, ch, D), dt),
        ]
        + [pltpu.SemaphoreType.DMA] * (2 * NB),
    )
    def gather_k(tbl_hbm, idx_hbm, out_hbm, idx_v, rows_v, *sems):
        gs = sems[:NB]
        ws = sems[NB:]
        wid = lax.axis_index("s") * 2 + lax.axis_index("c")
        base = wid * e_per_w

        def outer(i, carry):
            for b in range(NB):
                off = base + (i * NB + b) * ch

                @pl.when(i > 0)
                def _(off=off, b=b):
                    pltpu.make_async_copy(
                        rows_v.at[b], out_hbm.at[pl.ds(off, ch)], ws[b]
                    ).wait()

                pltpu.sync_copy(idx_hbm.at[pl.ds(off, ch)], idx_v.at[b])
                pltpu.async_copy(tbl_hbm.at[idx_v.at[b]], rows_v.at[b], gs[b])
            for b in range(NB):
                off = base + (i * NB + b) * ch
                pltpu.make_async_copy(tbl_hbm.at[idx_v.at[b]], rows_v.at[b], gs[b]).wait()
                pltpu.async_copy(rows_v.at[b], out_hbm.at[pl.ds(off, ch)], ws[b])
            return carry

        lax.fori_loop(0, n_outer, outer, 0)
        for b in range(NB):
            off = base + ((n_outer - 1) * NB + b) * ch
            pltpu.make_async_copy(rows_v.at[b], out_hbm.at[pl.ds(off, ch)], ws[b]).wait()

    return gather_k(table, col)


def _edge_z(fb, G, v2, W1b, b1r, W2, b2r, DEG):
    N, D = fb.shape
    E = G.shape[0]
    IN_DIM, HID = W1b.shape
    BN = 400
    BE = BN * DEG

    def body(f_ref, g_ref, v_ref, w1_ref, b1_ref, w2_ref, b2_ref, z_ref):
        f1 = jnp.broadcast_to(f_ref[...][:, None, :], (BN, DEG, D)).reshape(BE, D)
        temp = jnp.concatenate([f1, g_ref[...], v_ref[...]], axis=1)
        h = jnp.maximum(
            jnp.dot(temp, w1_ref[...], preferred_element_type=jnp.float32) + b1_ref[...],
            0.0,
        )
        z = jnp.dot(h, w2_ref[...], preferred_element_type=jnp.float32) + b2_ref[...]
        z_ref[...] = z.reshape(BN, DEG)

    return pl.pallas_call(
        body,
        grid=(N // BN,),
        in_specs=[
            pl.BlockSpec((BN, D), lambda i: (i, 0)),
            pl.BlockSpec((BE, D), lambda i: (i, 0)),
            pl.BlockSpec((BE, 1), lambda i: (i, 0)),
            pl.BlockSpec((IN_DIM, HID), lambda i: (0, 0)),
            pl.BlockSpec((1, HID), lambda i: (0, 0)),
            pl.BlockSpec((HID, 1), lambda i: (0, 0)),
            pl.BlockSpec((1, 1), lambda i: (0, 0)),
        ],
        out_specs=pl.BlockSpec((BN, DEG), lambda i: (i, 0)),
        out_shape=jax.ShapeDtypeStruct((N, DEG), jnp.float32),
    )(fb, G, v2, W1b, b1r, W2, b2r)


def _mask(zT, t2, km1):
    DEG, N = zT.shape

    def body(z_ref, t_ref, o_ref):
        zb = z_ref[...]
        ss = jnp.sum(zb * zb)
        zt = zb / jnp.maximum(jnp.sqrt(ss), 1e-12)
        m = jnp.max(zt, axis=0, keepdims=True)
        p = jnp.exp(zt - m)
        pi = p / jnp.sum(p, axis=0, keepdims=True)
        lo = jnp.log(pi) / t_ref[0, 0]
        m2 = jnp.max(lo, axis=0, keepdims=True)
        p2 = jnp.exp(lo - m2)
        y = p2 / jnp.sum(p2, axis=0, keepdims=True)
        cnt = jnp.zeros((DEG, N), jnp.float32)
        for i in range(DEG):
            cnt = cnt + (y[i : i + 1, :] > y).astype(jnp.float32)
        o_ref[...] = jnp.where(cnt <= km1, y, 0.0)

    return pl.pallas_call(
        body,
        grid=(1,),
        in_specs=[
            pl.BlockSpec((DEG, N), lambda i: (0, 0)),
            pl.BlockSpec((1, 1), lambda i: (0, 0)),
        ],
        out_specs=pl.BlockSpec((DEG, N), lambda i: (0, 0)),
        out_shape=jax.ShapeDtypeStruct((DEG, N), jnp.float32),
    )(zT, t2)


def kernel(features, indices, values, W1, b1, W2, b2, temperature):
    N, D = features.shape
    E = values.shape[0]
    DEG = E // N
    HID = W1.shape[1]
    K = max(int(round(DEG * 0.5)), 1)

    col = indices[1].astype(jnp.int32)
    v2 = values.reshape(E, 1)
    b1r = b1.reshape(1, HID)
    b2r = b2.reshape(1, 1)
    t2 = jnp.asarray(temperature, jnp.float32).reshape(1, 1)

    S = 2
    ns = N // S
    es = E // S
    zs = []
    for s in range(S):
        col_s = lax.slice_in_dim(col, s * es, (s + 1) * es)
        f_s = lax.slice_in_dim(features, s * ns, (s + 1) * ns)
        v_s = lax.slice_in_dim(v2, s * es, (s + 1) * es)
        G_s = _sc_gather(features, col_s)
        zs.append(_edge_z(f_s, G_s, v_s, W1, b1r, W2, b2r, DEG))
    z = jnp.concatenate(zs, axis=0)
    out = _mask(z.T, t2, float(K - 1))
    return out.T.reshape(E)


# speedup vs baseline: 1.2645x; 1.2645x over previous
import functools

import jax
import jax.numpy as jnp
from jax import lax
from jax.experimental import pallas as pl
from jax.experimental.pallas import tpu as pltpu
from jax.experimental.pallas import tpu_sc as plsc

_SC_WORKERS = 32
_GATHER_CHUNK = 80


def _sc_gather(table, col):
    E = col.shape[0]
    D = table.shape[1]
    dt = table.dtype
    e_per_w = E // _SC_WORKERS
    ch = _GATHER_CHUNK
    n_ch = e_per_w // ch
    mesh = plsc.VectorSubcoreMesh(core_axis_name="c", subcore_axis_name="s")

    NB = 5
    n_outer = n_ch // NB

    @functools.partial(
        pl.kernel,
        out_type=jax.ShapeDtypeStruct((E, D), dt),
        mesh=mesh,
        compiler_params=pltpu.CompilerParams(use_tc_tiling_on_sc=False),
        scratch_types=[
            pltpu.VMEM((NB, ch), jnp.int32),
            pltpu.VMEM((NB, ch, D), dt),
        ]
        + [pltpu.SemaphoreType.DMA] * (2 * NB),
    )
    def gather_k(tbl_hbm, idx_hbm, out_hbm, idx_v, rows_v, *sems):
        gs = sems[:NB]
        ws = sems[NB:]
        wid = lax.axis_index("s") * 2 + lax.axis_index("c")
        base = wid * e_per_w

        def outer(i, carry):
            for b in range(NB):
                off = base + (i * NB + b) * ch

                @pl.when(i > 0)
                def _(off=off, b=b):
                    pltpu.make_async_copy(
                        rows_v.at[b], out_hbm.at[pl.ds(off, ch)], ws[b]
                    ).wait()

                pltpu.sync_copy(idx_hbm.at[pl.ds(off, ch)], idx_v.at[b])
                pltpu.async_copy(tbl_hbm.at[idx_v.at[b]], rows_v.at[b], gs[b])
            for b in range(NB):
                off = base + (i * NB + b) * ch
                pltpu.make_async_copy(tbl_hbm.at[idx_v.at[b]], rows_v.at[b], gs[b]).wait()
                pltpu.async_copy(rows_v.at[b], out_hbm.at[pl.ds(off, ch)], ws[b])
            return carry

        lax.fori_loop(0, n_outer, outer, 0)
        for b in range(NB):
            off = base + ((n_outer - 1) * NB + b) * ch
            pltpu.make_async_copy(rows_v.at[b], out_hbm.at[pl.ds(off, ch)], ws[b]).wait()

    return gather_k(table, col)


def _edge_z(fb, G, v2, W1b, b1r, W2, b2r, DEG):
    N, D = fb.shape
    E = G.shape[0]
    IN_DIM, HID = W1b.shape
    BN = 400
    BE = BN * DEG

    def body(f_ref, g_ref, v_ref, w1_ref, b1_ref, w2_ref, b2_ref, z_ref):
        f1 = jnp.broadcast_to(f_ref[...][:, None, :], (BN, DEG, D)).reshape(BE, D)
        temp = jnp.concatenate([f1, g_ref[...], v_ref[...]], axis=1)
        h = jnp.maximum(
            jnp.dot(temp, w1_ref[...], preferred_element_type=jnp.float32) + b1_ref[...],
            0.0,
        )
        z = jnp.dot(h, w2_ref[...], preferred_element_type=jnp.float32) + b2_ref[...]
        z_ref[...] = z.reshape(BN, DEG)

    return pl.pallas_call(
        body,
        grid=(N // BN,),
        in_specs=[
            pl.BlockSpec((BN, D), lambda i: (i, 0)),
            pl.BlockSpec((BE, D), lambda i: (i, 0)),
            pl.BlockSpec((BE, 1), lambda i: (i, 0)),
            pl.BlockSpec((IN_DIM, HID), lambda i: (0, 0)),
            pl.BlockSpec((1, HID), lambda i: (0, 0)),
            pl.BlockSpec((HID, 1), lambda i: (0, 0)),
            pl.BlockSpec((1, 1), lambda i: (0, 0)),
        ],
        out_specs=pl.BlockSpec((BN, DEG), lambda i: (i, 0)),
        out_shape=jax.ShapeDtypeStruct((N, DEG), jnp.float32),
    )(fb, G, v2, W1b, b1r, W2, b2r)


def _mask(zT, t2, km1):
    DEG, N = zT.shape

    def body(z_ref, t_ref, o_ref):
        zb = z_ref[...]
        ss = jnp.sum(zb * zb)
        zt = zb / jnp.maximum(jnp.sqrt(ss), 1e-12)
        m = jnp.max(zt, axis=0, keepdims=True)
        p = jnp.exp(zt - m)
        pi = p / jnp.sum(p, axis=0, keepdims=True)
        lo = jnp.log(pi) / t_ref[0, 0]
        m2 = jnp.max(lo, axis=0, keepdims=True)
        p2 = jnp.exp(lo - m2)
        y = p2 / jnp.sum(p2, axis=0, keepdims=True)
        cnt = jnp.zeros((DEG, N), jnp.float32)
        for i in range(DEG):
            cnt = cnt + (y[i : i + 1, :] > y).astype(jnp.float32)
        o_ref[...] = jnp.where(cnt <= km1, y, 0.0)

    return pl.pallas_call(
        body,
        grid=(1,),
        in_specs=[
            pl.BlockSpec((DEG, N), lambda i: (0, 0)),
            pl.BlockSpec((1, 1), lambda i: (0, 0)),
        ],
        out_specs=pl.BlockSpec((DEG, N), lambda i: (0, 0)),
        out_shape=jax.ShapeDtypeStruct((DEG, N), jnp.float32),
    )(zT, t2)


def kernel(features, indices, values, W1, b1, W2, b2, temperature):
    N, D = features.shape
    E = values.shape[0]
    DEG = E // N
    HID = W1.shape[1]
    K = max(int(round(DEG * 0.5)), 1)

    col = indices[1].astype(jnp.int32)
    v2 = values.reshape(E, 1)
    b1r = b1.reshape(1, HID)
    b2r = b2.reshape(1, 1)
    t2 = jnp.asarray(temperature, jnp.float32).reshape(1, 1)

    G = _sc_gather(features, col)
    z = _edge_z(features, G, v2, W1, b1r, W2, b2r, DEG)
    out = _mask(z.T, t2, float(K - 1))
    return out.T.reshape(E)
